# baseline (speedup 1.0000x reference)
import functools

import jax
import jax.numpy as jnp
from jax import lax
from jax.experimental import pallas as pl
from jax.experimental.pallas import tpu as pltpu
from jax.experimental.pallas import tpu_sc as plsc

N_EXPERTS = 8
D_INPUT = 1024
M = 2048
TOPK = 8
THETA = 0.5
BATCH = 1024

B_TILE = 1024


def _expert_body(x_ref, keys_ref, masked_ref, fam_ref, y_ref, g_ref,
                 xn_ref, kn_ref):
    e = pl.program_id(0)

    @pl.when(e == 0)
    def _():
        xblk = x_ref[...]
        nrm = jnp.sqrt(jnp.sum(xblk * xblk, axis=1, keepdims=True)) + 1e-9
        xn_ref[...] = (xblk / nrm).astype(jnp.bfloat16)

    keys = keys_ref[0]
    nrm = jnp.sqrt(jnp.sum(keys * keys, axis=1, keepdims=True)) + 1e-9
    kn_ref[...] = (keys / nrm).astype(jnp.bfloat16)

    xn = xn_ref[...]
    halves = [
        lax.dot_general(
            xn, kn_ref[pl.ds(h * (M // 2), M // 2), :],
            (((1,), (1,)), ((), ())),
            preferred_element_type=jnp.float32,
        )
        for h in range(2)
    ]

    CH, CW = 16, M // 16
    SUB = 64

    def _sortnet(v, pairs):
        v = list(v)
        for i, j in pairs:
            hi = jnp.maximum(v[i], v[j])
            lo = jnp.minimum(v[i], v[j])
            v[i], v[j] = hi, lo
        return v

    _S8 = [(0, 1), (2, 3), (4, 5), (6, 7),
           (0, 2), (1, 3), (4, 6), (5, 7),
           (1, 2), (5, 6),
           (0, 4), (1, 5), (2, 6), (3, 7),
           (2, 4), (3, 5),
           (1, 2), (3, 4), (5, 6)]
    _B8 = [(0, 4), (1, 5), (2, 6), (3, 7),
           (0, 2), (1, 3), (4, 6), (5, 7),
           (0, 1), (2, 3), (4, 5), (6, 7)]

    tv_parts = [[] for _ in range(TOPK)]
    for rt in range(B_TILE // SUB):
        r0 = rt * SUB
        ch = [halves[c // 8][r0:r0 + SUB, (c % 8) * CW:(c % 8 + 1) * CW]
              for c in range(CH)]
        s1 = _sortnet(ch[:8], _S8)
        s2 = _sortnet(ch[8:], _S8)
        bit = [jnp.maximum(s1[i], s2[7 - i]) for i in range(8)]
        stk = _sortnet(bit, _B8)
        for i in range(TOPK):
            m = jnp.max(stk[0], axis=1, keepdims=True)
            tv_parts[i].append(m)
            if i < TOPK - 1:
                cond = stk[0] == m
                for d in range(7 - i):
                    stk[d] = jnp.where(cond, stk[d + 1], stk[d])

    tv = [jnp.concatenate(p, axis=0) for p in tv_parts]
    kth = tv[-1]
    fam = sum(tv) / TOPK
    exps = [jnp.exp(t - tv[0]) for t in tv]
    z = sum(exps)
    y = sum(ev * t for ev, t in zip(exps, tv)) / z
    g = (fam > THETA).astype(jnp.float32)

    for h in range(2):
        masked_ref[0, :, pl.ds(h * (M // 2), M // 2)] = jnp.where(
            halves[h] >= kth, halves[h], -jnp.inf)

    fam_ref[0] = fam
    y_ref[0] = y
    g_ref[0] = g


@jax.jit
def kernel(x, keys, v):
    masked, fam_e, y_e, g_e = pl.pallas_call(
        _expert_body,
        grid=(N_EXPERTS,),
        in_specs=[
            pl.BlockSpec((BATCH, D_INPUT), lambda e: (0, 0)),
            pl.BlockSpec((1, M, D_INPUT), lambda e: (e, 0, 0)),
        ],
        out_specs=[
            pl.BlockSpec((1, B_TILE, M), lambda e: (e, 0, 0)),
            pl.BlockSpec((1, BATCH, 1), lambda e: (e, 0, 0)),
            pl.BlockSpec((1, BATCH, 1), lambda e: (e, 0, 0)),
            pl.BlockSpec((1, BATCH, 1), lambda e: (e, 0, 0)),
        ],
        out_shape=[
            jax.ShapeDtypeStruct((N_EXPERTS, BATCH, M), jnp.float32),
            jax.ShapeDtypeStruct((N_EXPERTS, BATCH, 1), jnp.float32),
            jax.ShapeDtypeStruct((N_EXPERTS, BATCH, 1), jnp.float32),
            jax.ShapeDtypeStruct((N_EXPERTS, BATCH, 1), jnp.float32),
        ],
        scratch_shapes=[
            pltpu.VMEM((BATCH, D_INPUT), jnp.bfloat16),
            pltpu.VMEM((M, D_INPUT), jnp.bfloat16),
        ],
        compiler_params=pltpu.CompilerParams(
            vmem_limit_bytes=64 * 1024 * 1024),
    )(x, keys)

    winner, max_fam, y, g = _route_sc_call(
        fam_e.reshape(N_EXPERTS, BATCH),
        y_e.reshape(N_EXPERTS, BATCH),
        g_e.reshape(N_EXPERTS, BATCH))

    return (winner, max_fam, y, g, masked)


_NC, _NS, _L = 2, 16, 16
_NW = _NC * _NS
_BPW = BATCH // _NW


def _route_sc_body(fam_hbm, y_hbm, g_hbm, w_hbm, mf_hbm, yo_hbm, go_hbm,
                   fam_v, y_v, g_v, w_st, mf_st, yo_st, go_st):
    wid = lax.axis_index("s") * _NC + lax.axis_index("c")
    base = wid * _BPW
    for ee in range(N_EXPERTS):
        pltpu.sync_copy(fam_hbm.at[ee, pl.ds(base, _BPW)], fam_v.at[ee])
        pltpu.sync_copy(y_hbm.at[ee, pl.ds(base, _BPW)], y_v.at[ee])
        pltpu.sync_copy(g_hbm.at[ee, pl.ds(base, _BPW)], g_v.at[ee])
    for v in range(_BPW // _L):
        sl = pl.ds(v * _L, _L)
        wmax = fam_v[0, sl]
        widx = jnp.zeros((_L,), jnp.int32)
        ysel = y_v[0, sl]
        gsel = g_v[0, sl]
        for e in range(1, N_EXPERTS):
            f = fam_v[e, sl]
            m = f > wmax
            wmax = jnp.where(m, f, wmax)
            widx = jnp.where(m, jnp.full((_L,), e, jnp.int32), widx)
            ysel = jnp.where(m, y_v[e, sl], ysel)
            gsel = jnp.where(m, g_v[e, sl], gsel)
        w_st[sl] = widx
        mf_st[sl] = wmax
        yo_st[sl] = ysel
        go_st[sl] = gsel
    pltpu.sync_copy(w_st, w_hbm.at[pl.ds(base, _BPW)])
    pltpu.sync_copy(mf_st, mf_hbm.at[pl.ds(base, _BPW)])
    pltpu.sync_copy(yo_st, yo_hbm.at[pl.ds(base, _BPW)])
    pltpu.sync_copy(go_st, go_hbm.at[pl.ds(base, _BPW)])


def _route_sc_call(fam, y_e, g_e):
    return pl.kernel(
        _route_sc_body,
        mesh=plsc.VectorSubcoreMesh(core_axis_name="c", subcore_axis_name="s"),
        out_type=[
            jax.ShapeDtypeStruct((BATCH,), jnp.int32),
            jax.ShapeDtypeStruct((BATCH,), jnp.float32),
            jax.ShapeDtypeStruct((BATCH,), jnp.float32),
            jax.ShapeDtypeStruct((BATCH,), jnp.float32),
        ],
        scratch_types=[
            pltpu.VMEM((N_EXPERTS, _BPW), jnp.float32),
            pltpu.VMEM((N_EXPERTS, _BPW), jnp.float32),
            pltpu.VMEM((N_EXPERTS, _BPW), jnp.float32),
            pltpu.VMEM((_BPW,), jnp.int32),
            pltpu.VMEM((_BPW,), jnp.float32),
            pltpu.VMEM((_BPW,), jnp.float32),
            pltpu.VMEM((_BPW,), jnp.float32),
        ],
    )(fam, y_e, g_e)


# speedup vs baseline: 1.2871x; 1.2871x over previous
import functools

import jax
import jax.numpy as jnp
from jax import lax
from jax.experimental import pallas as pl
from jax.experimental.pallas import tpu as pltpu

N_EXPERTS = 8
D_INPUT = 1024
M = 2048
TOPK = 8
THETA = 0.5
BATCH = 1024

B_TILE = 1024


def _expert_body(x_ref, keys_ref, masked_ref, w_ref, mf_ref, yo_ref, go_ref,
                 xn_ref, kn_ref):
    e = pl.program_id(0)

    @pl.when(e == 0)
    def _():
        xblk = x_ref[...]
        nrm = jnp.sqrt(jnp.sum(xblk * xblk, axis=1, keepdims=True)) + 1e-9
        xn_ref[...] = (xblk / nrm).astype(jnp.bfloat16)

    keys = keys_ref[0]
    nrm = jnp.sqrt(jnp.sum(keys * keys, axis=1, keepdims=True)) + 1e-9
    kn_ref[...] = (keys / nrm).astype(jnp.bfloat16)

    xn = xn_ref[...]
    halves = [
        lax.dot_general(
            xn, kn_ref[pl.ds(h * (M // 2), M // 2), :],
            (((1,), (1,)), ((), ())),
            preferred_element_type=jnp.float32,
        )
        for h in range(2)
    ]

    CH, CW = 16, M // 16
    SUB = 64

    def _sortnet(v, pairs):
        v = list(v)
        for i, j in pairs:
            hi = jnp.maximum(v[i], v[j])
            lo = jnp.minimum(v[i], v[j])
            v[i], v[j] = hi, lo
        return v

    _S8 = [(0, 1), (2, 3), (4, 5), (6, 7),
           (0, 2), (1, 3), (4, 6), (5, 7),
           (1, 2), (5, 6),
           (0, 4), (1, 5), (2, 6), (3, 7),
           (2, 4), (3, 5),
           (1, 2), (3, 4), (5, 6)]
    _B8 = [(0, 4), (1, 5), (2, 6), (3, 7),
           (0, 2), (1, 3), (4, 6), (5, 7),
           (0, 1), (2, 3), (4, 5), (6, 7)]

    tv_parts = [[] for _ in range(TOPK)]
    for rt in range(B_TILE // SUB):
        r0 = rt * SUB
        ch = [halves[c // 8][r0:r0 + SUB, (c % 8) * CW:(c % 8 + 1) * CW]
              for c in range(CH)]
        s1 = _sortnet(ch[:8], _S8)
        s2 = _sortnet(ch[8:], _S8)
        bit = [jnp.maximum(s1[i], s2[7 - i]) for i in range(8)]
        stk = _sortnet(bit, _B8)
        for i in range(TOPK):
            m = jnp.max(stk[0], axis=1, keepdims=True)
            tv_parts[i].append(m)
            if i < TOPK - 1:
                cond = stk[0] == m
                for d in range(7 - i):
                    stk[d] = jnp.where(cond, stk[d + 1], stk[d])

    tv = [jnp.concatenate(p, axis=0) for p in tv_parts]
    kth = tv[-1]
    fam = sum(tv) / TOPK
    exps = [jnp.exp(t - tv[0]) for t in tv]
    z = sum(exps)
    y = sum(ev * t for ev, t in zip(exps, tv)) / z
    g = (fam > THETA).astype(jnp.float32)

    for h in range(2):
        masked_ref[0, :, pl.ds(h * (M // 2), M // 2)] = jnp.where(
            halves[h] >= kth, halves[h], -jnp.inf)

    @pl.when(e == 0)
    def _():
        mf_ref[...] = fam
        w_ref[...] = jnp.zeros((B_TILE, 1), jnp.int32)
        yo_ref[...] = y
        go_ref[...] = g

    @pl.when(e > 0)
    def _():
        wm = mf_ref[...]
        cond = fam > wm
        mf_ref[...] = jnp.where(cond, fam, wm)
        w_ref[...] = jnp.where(cond, e, w_ref[...])
        yo_ref[...] = jnp.where(cond, y, yo_ref[...])
        go_ref[...] = jnp.where(cond, g, go_ref[...])


@jax.jit
def kernel(x, keys, v):
    masked, winner, max_fam, y, g = pl.pallas_call(
        _expert_body,
        grid=(N_EXPERTS,),
        in_specs=[
            pl.BlockSpec((BATCH, D_INPUT), lambda e: (0, 0)),
            pl.BlockSpec((1, M, D_INPUT), lambda e: (e, 0, 0)),
        ],
        out_specs=[
            pl.BlockSpec((1, B_TILE, M), lambda e: (e, 0, 0)),
            pl.BlockSpec((BATCH, 1), lambda e: (0, 0)),
            pl.BlockSpec((BATCH, 1), lambda e: (0, 0)),
            pl.BlockSpec((BATCH, 1), lambda e: (0, 0)),
            pl.BlockSpec((BATCH, 1), lambda e: (0, 0)),
        ],
        out_shape=[
            jax.ShapeDtypeStruct((N_EXPERTS, BATCH, M), jnp.float32),
            jax.ShapeDtypeStruct((BATCH, 1), jnp.int32),
            jax.ShapeDtypeStruct((BATCH, 1), jnp.float32),
            jax.ShapeDtypeStruct((BATCH, 1), jnp.float32),
            jax.ShapeDtypeStruct((BATCH, 1), jnp.float32),
        ],
        scratch_shapes=[
            pltpu.VMEM((BATCH, D_INPUT), jnp.bfloat16),
            pltpu.VMEM((M, D_INPUT), jnp.bfloat16),
        ],
    )(x, keys)

    return (winner.reshape(BATCH), max_fam.reshape(BATCH),
            y.reshape(BATCH), g.reshape(BATCH), masked)
